# baseline (speedup 1.0000x reference)
import functools

import jax
import jax.numpy as jnp
from jax import lax
from jax.experimental import pallas as pl
from jax.experimental.pallas import tpu as pltpu
from jax.experimental.pallas import tpu_sc as plsc

_N = 10000
_E = 320000
_DIM = 128
_NET = 4
_NSTEPS = 4

_NC = 2
_NS = 16
_NW = _NC * _NS
_EPW = _E // _NW
_C = 80
_NCH = _EPW // _C
_RPT = 624
_REM = _N - _NS * _RPT


def _msg_body(h_ref, w_ref, b_ref, o_ref):
    o_ref[...] = (
        jnp.dot(h_ref[...], w_ref[...], preferred_element_type=jnp.float32)
        + b_ref[...]
    )


def _msg(h, w_t, b2d):
    rb = 1000
    return pl.pallas_call(
        _msg_body,
        grid=(_N // rb,),
        in_specs=[
            pl.BlockSpec((rb, _DIM), lambda i: (i, 0)),
            pl.BlockSpec((_DIM, _NET * _DIM), lambda i: (0, 0)),
            pl.BlockSpec((1, _NET * _DIM), lambda i: (0, 0)),
        ],
        out_specs=pl.BlockSpec((rb, _NET * _DIM), lambda i: (i, 0)),
        out_shape=jax.ShapeDtypeStruct((_N, _NET * _DIM), jnp.float32),
    )(h, w_t, b2d)


@functools.partial(
    pl.kernel,
    out_type=jax.ShapeDtypeStruct((_NC, _N, _DIM), jnp.float32),
    mesh=plsc.VectorSubcoreMesh(core_axis_name="c", subcore_axis_name="s"),
    scratch_types=[
        pltpu.VMEM((1, _C), jnp.int32),
        pltpu.VMEM((1, _C), jnp.int32),
        pltpu.VMEM((1, _C), jnp.int32),
        pltpu.VMEM((1, _C), jnp.int32),
        pltpu.VMEM((2, _C), jnp.int32),
        pltpu.VMEM((2, _C), jnp.int32),
        pltpu.VMEM((2, _C), jnp.int32),
        pltpu.VMEM((2, _C), jnp.int32),
        pltpu.VMEM((_C, _DIM), jnp.float32),
        pltpu.VMEM((_C, _DIM), jnp.float32),
        pltpu.VMEM((_C, _DIM), jnp.float32),
        pltpu.VMEM((_C, _DIM), jnp.float32),
        pltpu.VMEM_SHARED((_N, _DIM), jnp.float32),
    ] + [pltpu.SemaphoreType.DMA] * 12,
)
def _sc_agg(m_hbm, idx_hbm, z_hbm, out_hbm,
            pb0, pb1, pb2, pb3, sdb0, sdb1, sdb2, sdb3,
            rb0, rb1, rb2, rb3, aggsh,
            gs0, gs1, gs2, gs3, ss0, ss1, ss2, ss3, is0, is1, is2, is3):
    cid = lax.axis_index("c")
    sid = lax.axis_index("s")
    wid = sid * _NC + cid

    def _unpack(pb, sdb):
        for k in range(_C // 16):
            x = pb[0, pl.ds(k * 16, 16)]
            sdb[0, pl.ds(k * 16, 16)] = lax.bitwise_and(x, 0xFFFF)
            sdb[1, pl.ds(k * 16, 16)] = lax.shift_right_logical(x, 16)

    pltpu.sync_copy(z_hbm, aggsh.at[pl.ds(sid * _RPT, _RPT)])

    @pl.when(sid == 0)
    def _():
        pltpu.sync_copy(
            z_hbm.at[pl.ds(0, _REM)], aggsh.at[pl.ds(_NS * _RPT, _REM)]
        )

    slots = (
        (pb0, sdb0, rb0, gs0, ss0, is0),
        (pb1, sdb1, rb1, gs1, ss1, is1),
        (pb2, sdb2, rb2, gs2, ss2, is2),
        (pb3, sdb3, rb3, gs3, ss3, is3),
    )

    for b in range(4):
        pltpu.async_copy(idx_hbm.at[wid, b], slots[b][0], slots[b][5])
    for b in range(2):
        pb, sdb, rb, gs, ss, isem = slots[b]
        pltpu.make_async_copy(idx_hbm.at[wid, b], pb, isem).wait()
        _unpack(pb, sdb)
        pltpu.async_copy(m_hbm.at[sdb.at[0]], rb, gs)

    plsc.subcore_barrier()

    def _quad(i4, carry):
        k0 = i4 * 4
        for r in range(4):
            k = k0 + r
            pb, sdb, rb, gs, ss, isem = slots[r]
            pb2, sdb2, rb2, gs2, ss2_, is2_ = slots[(r + 2) % 4]
            pltpu.make_async_copy(m_hbm.at[sdb.at[0]], rb, gs).wait()
            pltpu.async_copy(rb, aggsh.at[sdb.at[1]], ss, add=True)

            @pl.when(k + 4 < _NCH)
            def _():
                pltpu.async_copy(idx_hbm.at[wid, k + 4], pb, isem)

            @pl.when(k >= 2)
            def _():
                pltpu.make_async_copy(rb2, aggsh.at[sdb2.at[1]], ss2_).wait()

            @pl.when(k + 2 < _NCH)
            def _():
                pltpu.make_async_copy(idx_hbm.at[wid, k + 2], pb2, is2_).wait()
                _unpack(pb2, sdb2)
                pltpu.async_copy(m_hbm.at[sdb2.at[0]], rb2, gs2)
        return carry

    _nquad = (_NCH - 1) // 4
    assert _nquad * 4 + 1 == _NCH
    lax.fori_loop(0, _nquad, _quad, 0)
    pltpu.make_async_copy(m_hbm.at[sdb0.at[0]], rb0, gs0).wait()
    pltpu.async_copy(rb0, aggsh.at[sdb0.at[1]], ss0, add=True)
    pltpu.make_async_copy(rb2, aggsh.at[sdb2.at[1]], ss2).wait()
    pltpu.make_async_copy(rb3, aggsh.at[sdb3.at[1]], ss3).wait()
    pltpu.make_async_copy(rb0, aggsh.at[sdb0.at[1]], ss0).wait()

    plsc.subcore_barrier()
    pltpu.sync_copy(
        aggsh.at[pl.ds(sid * _RPT, _RPT)],
        out_hbm.at[cid, pl.ds(sid * _RPT, _RPT)],
    )

    @pl.when(sid == 0)
    def _():
        pltpu.sync_copy(
            aggsh.at[pl.ds(_NS * _RPT, _REM)],
            out_hbm.at[cid, pl.ds(_NS * _RPT, _REM)],
        )


def _gru_msg_body(a_ref, h_ref, wih_ref, whh_ref, bih_ref, bhh_ref,
                  wm_ref, bm_ref, o_ref, m_ref):
    agg = a_ref[0] + a_ref[1]
    h = h_ref[...]
    gi = (
        jnp.dot(agg, wih_ref[...], preferred_element_type=jnp.float32)
        + bih_ref[...]
    )
    gh = (
        jnp.dot(h, whh_ref[...], preferred_element_type=jnp.float32)
        + bhh_ref[...]
    )
    rg = jax.nn.sigmoid(gi[:, :_DIM] + gh[:, :_DIM])
    zg = jax.nn.sigmoid(gi[:, _DIM:2 * _DIM] + gh[:, _DIM:2 * _DIM])
    ng = jnp.tanh(gi[:, 2 * _DIM:] + rg * gh[:, 2 * _DIM:])
    h_new = (1.0 - zg) * ng + zg * h
    o_ref[...] = h_new
    m_ref[...] = (
        jnp.dot(h_new, wm_ref[...], preferred_element_type=jnp.float32)
        + bm_ref[...]
    )


def _gru_msg(agg2, h, wih_t, whh_t, bih2d, bhh2d, wm_t, bm2d):
    rb = 1000
    return pl.pallas_call(
        _gru_msg_body,
        grid=(_N // rb,),
        in_specs=[
            pl.BlockSpec((_NC, rb, _DIM), lambda i: (0, i, 0)),
            pl.BlockSpec((rb, _DIM), lambda i: (i, 0)),
            pl.BlockSpec((_DIM, 3 * _DIM), lambda i: (0, 0)),
            pl.BlockSpec((_DIM, 3 * _DIM), lambda i: (0, 0)),
            pl.BlockSpec((1, 3 * _DIM), lambda i: (0, 0)),
            pl.BlockSpec((1, 3 * _DIM), lambda i: (0, 0)),
            pl.BlockSpec((_DIM, _NET * _DIM), lambda i: (0, 0)),
            pl.BlockSpec((1, _NET * _DIM), lambda i: (0, 0)),
        ],
        out_specs=[
            pl.BlockSpec((rb, _DIM), lambda i: (i, 0)),
            pl.BlockSpec((rb, _NET * _DIM), lambda i: (i, 0)),
        ],
        out_shape=[
            jax.ShapeDtypeStruct((_N, _DIM), jnp.float32),
            jax.ShapeDtypeStruct((_N, _NET * _DIM), jnp.float32),
        ],
    )(agg2, h, wih_t, whh_t, bih2d, bhh2d, wm_t, bm2d)


def _gru_body(a_ref, h_ref, wih_ref, whh_ref, bih_ref, bhh_ref, o_ref):
    agg = a_ref[0] + a_ref[1]
    h = h_ref[...]
    gi = (
        jnp.dot(agg, wih_ref[...], preferred_element_type=jnp.float32)
        + bih_ref[...]
    )
    gh = (
        jnp.dot(h, whh_ref[...], preferred_element_type=jnp.float32)
        + bhh_ref[...]
    )
    rg = jax.nn.sigmoid(gi[:, :_DIM] + gh[:, :_DIM])
    zg = jax.nn.sigmoid(gi[:, _DIM:2 * _DIM] + gh[:, _DIM:2 * _DIM])
    ng = jnp.tanh(gi[:, 2 * _DIM:] + rg * gh[:, 2 * _DIM:])
    o_ref[...] = (1.0 - zg) * ng + zg * h


def _gru(agg2, h, wih_t, whh_t, bih2d, bhh2d):
    rb = 1000
    return pl.pallas_call(
        _gru_body,
        grid=(_N // rb,),
        in_specs=[
            pl.BlockSpec((_NC, rb, _DIM), lambda i: (0, i, 0)),
            pl.BlockSpec((rb, _DIM), lambda i: (i, 0)),
            pl.BlockSpec((_DIM, 3 * _DIM), lambda i: (0, 0)),
            pl.BlockSpec((_DIM, 3 * _DIM), lambda i: (0, 0)),
            pl.BlockSpec((1, 3 * _DIM), lambda i: (0, 0)),
            pl.BlockSpec((1, 3 * _DIM), lambda i: (0, 0)),
        ],
        out_specs=pl.BlockSpec((rb, _DIM), lambda i: (i, 0)),
        out_shape=jax.ShapeDtypeStruct((_N, _DIM), jnp.float32),
    )(agg2, h, wih_t, whh_t, bih2d, bhh2d)


def kernel(embedding, edges, W_msg, b_msg, W_ih, W_hh, b_ih, b_hh):
    wm_t = W_msg.T
    wih_t = W_ih.T
    whh_t = W_hh.T
    bm2d = b_msg.reshape(1, _NET * _DIM)
    bih2d = b_ih.reshape(1, 3 * _DIM)
    bhh2d = b_hh.reshape(1, 3 * _DIM)

    e = edges.astype(jnp.int32)
    src = e[:, 0] * _NET + e[:, 2]
    dst = e[:, 1]
    packed = jnp.bitwise_or(jnp.left_shift(dst, 16), src).reshape(
        _NW, _NCH, 1, _C)
    zrows = jnp.zeros((_RPT, _DIM), jnp.float32)

    h = embedding
    m = _msg(h, wm_t, bm2d)
    for step in range(_NSTEPS):
        agg2 = _sc_agg(m.reshape(_N * _NET, _DIM), packed, zrows)
        if step < _NSTEPS - 1:
            h, m = _gru_msg(agg2, h, wih_t, whh_t, bih2d, bhh2d, wm_t, bm2d)
        else:
            h = _gru(agg2, h, wih_t, whh_t, bih2d, bhh2d)
    return h
